# Initial kernel scaffold; baseline (speedup 1.0000x reference)
import jax
import jax.numpy as jnp
from jax import lax
from jax.experimental import pallas as pl
from jax.experimental.pallas import tpu as pltpu
from jax.experimental.pallas import tpu_sc as plsc

EPS_DIV = 1e-06
LANES = 16
NUM_WORKERS = 32


def _sc_recip_gather(scale_p, idx_p, g_pad):
    b_per_w = g_pad // NUM_WORKERS
    mesh = plsc.VectorSubcoreMesh(core_axis_name="c", subcore_axis_name="s")

    def body(scale_hbm, idx_hbm, out_hbm, table_v, idx_v, out_v):
        nc = lax.axis_size("c")
        wid = lax.axis_index("s") * nc + lax.axis_index("c")
        base = wid * b_per_w
        pltpu.sync_copy(scale_hbm, table_v)
        pltpu.sync_copy(idx_hbm.at[pl.ds(base, b_per_w)], idx_v)

        def step(i, carry):
            off = pl.multiple_of(i * LANES, LANES)
            iv = idx_v[pl.ds(off, LANES)]
            v = plsc.load_gather(table_v, [iv])
            out_v[pl.ds(off, LANES)] = 1.0 / (v + EPS_DIV)
            return carry

        lax.fori_loop(0, b_per_w // LANES, step, 0)
        pltpu.sync_copy(out_v, out_hbm.at[pl.ds(base, b_per_w)])

    fn = pl.kernel(
        body,
        out_type=jax.ShapeDtypeStruct((g_pad,), jnp.float32),
        mesh=mesh,
        scratch_types=[
            pltpu.VMEM((g_pad,), jnp.float32),
            pltpu.VMEM((b_per_w,), jnp.int32),
            pltpu.VMEM((b_per_w,), jnp.float32),
        ],
    )
    return fn(scale_p, idx_p)


def _tc_scale_mul(x_ng, recip_row, block_rows):
    n, g = x_ng.shape

    def body(x_ref, r_ref, o_ref):
        o_ref[...] = x_ref[...] * r_ref[...]

    return pl.pallas_call(
        body,
        grid=(pl.cdiv(n, block_rows),),
        in_specs=[
            pl.BlockSpec((block_rows, g), lambda i: (i, 0)),
            pl.BlockSpec((1, g), lambda i: (0, 0)),
        ],
        out_specs=pl.BlockSpec((block_rows, g), lambda i: (i, 0)),
        out_shape=jax.ShapeDtypeStruct((n, g), jnp.float32),
    )(x_ng, recip_row)


@jax.jit
def kernel(x_ng, scale_g, idx):
    n, g = x_ng.shape
    chunk = NUM_WORKERS * LANES
    g_pad = ((g + chunk - 1) // chunk) * chunk
    idx_p = jnp.pad(idx, (0, g_pad - g))
    scale_p = jnp.pad(scale_g, (0, g_pad - scale_g.shape[0]))
    recip = _sc_recip_gather(scale_p, idx_p, g_pad)
    recip_row = recip[:g].reshape(1, g)
    return _tc_scale_mul(x_ng, recip_row, block_rows=16)


# speedup vs baseline: 2.1538x; 2.1538x over previous
import jax
import jax.numpy as jnp
from jax import lax
from jax.experimental import pallas as pl
from jax.experimental.pallas import tpu as pltpu
from jax.experimental.pallas import tpu_sc as plsc

EPS_DIV = 1e-06
LANES = 16
NUM_WORKERS = 32


def _sc_recip_gather(scale_p, idx_p, g_pad):
    b_per_w = g_pad // NUM_WORKERS
    mesh = plsc.VectorSubcoreMesh(core_axis_name="c", subcore_axis_name="s")

    def body(scale_hbm, idx_hbm, out_hbm, table_v, idx_v, out_v):
        nc = lax.axis_size("c")
        wid = lax.axis_index("s") * nc + lax.axis_index("c")
        base = wid * b_per_w
        pltpu.sync_copy(scale_hbm, table_v)
        pltpu.sync_copy(idx_hbm.at[pl.ds(base, b_per_w)], idx_v)

        def step(i, carry):
            off = pl.multiple_of(i * LANES, LANES)
            iv = idx_v[pl.ds(off, LANES)]
            v = plsc.load_gather(table_v, [iv])
            out_v[pl.ds(off, LANES)] = 1.0 / (v + EPS_DIV)
            return carry

        lax.fori_loop(0, b_per_w // LANES, step, 0)
        pltpu.sync_copy(out_v, out_hbm.at[pl.ds(base, b_per_w)])

    fn = pl.kernel(
        body,
        out_type=jax.ShapeDtypeStruct((g_pad,), jnp.float32),
        mesh=mesh,
        compiler_params=pltpu.CompilerParams(needs_layout_passes=False),
        scratch_types=[
            pltpu.VMEM((g_pad,), jnp.float32),
            pltpu.VMEM((b_per_w,), jnp.int32),
            pltpu.VMEM((b_per_w,), jnp.float32),
        ],
    )
    return fn(scale_p, idx_p)


def _tc_scale_mul(x_ng, recip_row, block_rows):
    n, g = x_ng.shape

    def body(x_ref, r_ref, o_ref):
        o_ref[...] = x_ref[...] * r_ref[...]

    return pl.pallas_call(
        body,
        grid=(pl.cdiv(n, block_rows),),
        in_specs=[
            pl.BlockSpec((block_rows, g), lambda i: (i, 0)),
            pl.BlockSpec((1, g), lambda i: (0, 0)),
        ],
        out_specs=pl.BlockSpec((block_rows, g), lambda i: (i, 0)),
        out_shape=jax.ShapeDtypeStruct((n, g), jnp.float32),
    )(x_ng, recip_row)


@jax.jit
def kernel(x_ng, scale_g, idx):
    n, g = x_ng.shape
    chunk = NUM_WORKERS * LANES
    g_pad = ((g + chunk - 1) // chunk) * chunk
    idx_p = jnp.pad(idx, (0, g_pad - g))
    scale_p = jnp.pad(scale_g, (0, g_pad - scale_g.shape[0]))
    recip = _sc_recip_gather(scale_p, idx_p, g_pad)
    recip_row = recip[:g].reshape(1, g)
    return _tc_scale_mul(x_ng, recip_row, block_rows=16)


# speedup vs baseline: 2.1887x; 1.0162x over previous
import jax
import jax.numpy as jnp
from jax import lax
from jax.experimental import pallas as pl
from jax.experimental.pallas import tpu as pltpu
from jax.experimental.pallas import tpu_sc as plsc

EPS_DIV = 1e-06
LANES = 16
NUM_WORKERS = 32


def _sc_recip_gather(scale_p, idx_p, g_pad):
    b_per_w = g_pad // NUM_WORKERS
    mesh = plsc.VectorSubcoreMesh(core_axis_name="c", subcore_axis_name="s")

    def body(scale_hbm, idx_hbm, out_hbm, table_v, idx_v, out_v):
        nc = lax.axis_size("c")
        wid = lax.axis_index("s") * nc + lax.axis_index("c")
        base = wid * b_per_w
        pltpu.sync_copy(scale_hbm, table_v)
        pltpu.sync_copy(idx_hbm.at[pl.ds(base, b_per_w)], idx_v)

        def step(i, carry):
            off = pl.multiple_of(i * LANES, LANES)
            iv = idx_v[pl.ds(off, LANES)]
            v = plsc.load_gather(table_v, [iv])
            out_v[pl.ds(off, LANES)] = 1.0 / (v + EPS_DIV)
            return carry

        lax.fori_loop(0, b_per_w // LANES, step, 0)
        pltpu.sync_copy(out_v, out_hbm.at[0, pl.ds(base, b_per_w)])

    fn = pl.kernel(
        body,
        out_type=jax.ShapeDtypeStruct((1, g_pad), jnp.float32),
        mesh=mesh,
        compiler_params=pltpu.CompilerParams(needs_layout_passes=False),
        scratch_types=[
            pltpu.VMEM((scale_p.shape[0],), jnp.float32),
            pltpu.VMEM((b_per_w,), jnp.int32),
            pltpu.VMEM((b_per_w,), jnp.float32),
        ],
    )
    return fn(scale_p, idx_p)


def _tc_scale_mul(x_ng, recip_row, block_rows):
    n, g = x_ng.shape

    g_pad = recip_row.shape[1]

    def body(x_ref, r_ref, o_ref):
        o_ref[...] = x_ref[...] * r_ref[...][:, :g]

    return pl.pallas_call(
        body,
        grid=(pl.cdiv(n, block_rows),),
        in_specs=[
            pl.BlockSpec((block_rows, g), lambda i: (i, 0)),
            pl.BlockSpec((1, g_pad), lambda i: (0, 0)),
        ],
        out_specs=pl.BlockSpec((block_rows, g), lambda i: (i, 0)),
        out_shape=jax.ShapeDtypeStruct((n, g), jnp.float32),
    )(x_ng, recip_row)


@jax.jit
def kernel(x_ng, scale_g, idx):
    n, g = x_ng.shape
    chunk = NUM_WORKERS * LANES
    g_pad = ((g + chunk - 1) // chunk) * chunk
    idx_p = jnp.pad(idx, (0, g_pad - g))
    recip_row = _sc_recip_gather(scale_g, idx_p, g_pad)
    return _tc_scale_mul(x_ng, recip_row, block_rows=16)


# speedup vs baseline: 2.3098x; 1.0553x over previous
import jax
import jax.numpy as jnp
from jax import lax
from jax.experimental import pallas as pl
from jax.experimental.pallas import tpu as pltpu
from jax.experimental.pallas import tpu_sc as plsc

EPS_DIV = 1e-06
LANES = 16
NUM_WORKERS = 32


def _sc_recip_gather(scale_p, idx_p, g_pad):
    b_per_w = g_pad // NUM_WORKERS
    mesh = plsc.VectorSubcoreMesh(core_axis_name="c", subcore_axis_name="s")

    def body(scale_hbm, idx_hbm, out_hbm, table_v, idx_v, out_v):
        nc = lax.axis_size("c")
        wid = lax.axis_index("s") * nc + lax.axis_index("c")
        base = wid * b_per_w
        pltpu.sync_copy(scale_hbm, table_v)
        pltpu.sync_copy(idx_hbm.at[pl.ds(base, b_per_w)], idx_v)

        def step(i, carry):
            off = pl.multiple_of(i * LANES, LANES)
            iv = idx_v[pl.ds(off, LANES)]
            v = plsc.load_gather(table_v, [iv])
            out_v[pl.ds(off, LANES)] = 1.0 / (v + EPS_DIV)
            return carry

        lax.fori_loop(0, b_per_w // LANES, step, 0)
        pltpu.sync_copy(out_v, out_hbm.at[0, pl.ds(base, b_per_w)])

    fn = pl.kernel(
        body,
        out_type=jax.ShapeDtypeStruct((1, g_pad), jnp.float32),
        mesh=mesh,
        compiler_params=pltpu.CompilerParams(needs_layout_passes=False),
        scratch_types=[
            pltpu.VMEM((scale_p.shape[0],), jnp.float32),
            pltpu.VMEM((b_per_w,), jnp.int32),
            pltpu.VMEM((b_per_w,), jnp.float32),
        ],
    )
    return fn(scale_p, idx_p)


def _tc_scale_mul(x_ng, recip_row, block_rows):
    n, g = x_ng.shape

    g_pad = recip_row.shape[1]

    def body(x_ref, r_ref, o_ref):
        o_ref[...] = x_ref[...] * r_ref[...][:, :g]

    return pl.pallas_call(
        body,
        grid=(pl.cdiv(n, block_rows),),
        in_specs=[
            pl.BlockSpec((block_rows, g), lambda i: (i, 0)),
            pl.BlockSpec((1, g_pad), lambda i: (0, 0)),
        ],
        out_specs=pl.BlockSpec((block_rows, g), lambda i: (i, 0)),
        out_shape=jax.ShapeDtypeStruct((n, g), jnp.float32),
    )(x_ng, recip_row)


@jax.jit
def kernel(x_ng, scale_g, idx):
    n, g = x_ng.shape
    chunk = NUM_WORKERS * LANES
    g_pad = ((g + chunk - 1) // chunk) * chunk
    idx_p = jnp.pad(idx, (0, g_pad - g))
    recip_row = _sc_recip_gather(scale_g, idx_p, g_pad)
    return _tc_scale_mul(x_ng, recip_row, block_rows=32)


# speedup vs baseline: 2.3405x; 1.0133x over previous
import jax
import jax.numpy as jnp
from jax import lax
from jax.experimental import pallas as pl
from jax.experimental.pallas import tpu as pltpu
from jax.experimental.pallas import tpu_sc as plsc

EPS_DIV = 1e-06
LANES = 16
NUM_WORKERS = 32


def _sc_recip_gather(scale_p, idx_p, g_pad):
    b_per_w = g_pad // NUM_WORKERS
    mesh = plsc.VectorSubcoreMesh(core_axis_name="c", subcore_axis_name="s")

    def body(scale_hbm, idx_hbm, out_hbm, table_v, idx_v, out_v):
        nc = lax.axis_size("c")
        wid = lax.axis_index("s") * nc + lax.axis_index("c")
        base = wid * b_per_w
        pltpu.sync_copy(scale_hbm, table_v)
        pltpu.sync_copy(idx_hbm.at[pl.ds(base, b_per_w)], idx_v)

        def step(i, carry):
            off = pl.multiple_of(i * LANES, LANES)
            iv = idx_v[pl.ds(off, LANES)]
            v = plsc.load_gather(table_v, [iv])
            out_v[pl.ds(off, LANES)] = 1.0 / (v + EPS_DIV)
            return carry

        lax.fori_loop(0, b_per_w // LANES, step, 0)
        pltpu.sync_copy(out_v, out_hbm.at[0, pl.ds(base, b_per_w)])

    fn = pl.kernel(
        body,
        out_type=jax.ShapeDtypeStruct((1, g_pad), jnp.float32),
        mesh=mesh,
        compiler_params=pltpu.CompilerParams(needs_layout_passes=False),
        scratch_types=[
            pltpu.VMEM((scale_p.shape[0],), jnp.float32),
            pltpu.VMEM((b_per_w,), jnp.int32),
            pltpu.VMEM((b_per_w,), jnp.float32),
        ],
    )
    return fn(scale_p, idx_p)


def _tc_scale_mul(x_ng, recip_row, block_rows):
    n, g = x_ng.shape

    g_pad = recip_row.shape[1]

    def body(x_ref, r_ref, o_ref):
        o_ref[...] = x_ref[...] * r_ref[...][:, :g]

    return pl.pallas_call(
        body,
        grid=(pl.cdiv(n, block_rows),),
        in_specs=[
            pl.BlockSpec((block_rows, g), lambda i: (i, 0)),
            pl.BlockSpec((1, g_pad), lambda i: (0, 0)),
        ],
        out_specs=pl.BlockSpec((block_rows, g), lambda i: (i, 0)),
        out_shape=jax.ShapeDtypeStruct((n, g), jnp.float32),
    )(x_ng, recip_row)


@jax.jit
def kernel(x_ng, scale_g, idx):
    n, g = x_ng.shape
    chunk = NUM_WORKERS * LANES
    g_pad = ((g + chunk - 1) // chunk) * chunk
    idx_p = jnp.pad(idx, (0, g_pad - g))
    recip_row = _sc_recip_gather(scale_g, idx_p, g_pad)
    return _tc_scale_mul(x_ng, recip_row, block_rows=64)


# speedup vs baseline: 2.3763x; 1.0153x over previous
import jax
import jax.numpy as jnp
from jax import lax
from jax.experimental import pallas as pl
from jax.experimental.pallas import tpu as pltpu
from jax.experimental.pallas import tpu_sc as plsc

EPS_DIV = 1e-06
LANES = 16
NUM_WORKERS = 32


def _sc_recip_gather(scale_p, idx_p, g_pad):
    b_per_w = g_pad // NUM_WORKERS
    mesh = plsc.VectorSubcoreMesh(core_axis_name="c", subcore_axis_name="s")

    def body(scale_hbm, idx_hbm, out_hbm, table_v, idx_v, out_v):
        nc = lax.axis_size("c")
        wid = lax.axis_index("s") * nc + lax.axis_index("c")
        base = wid * b_per_w
        pltpu.sync_copy(scale_hbm, table_v)
        pltpu.sync_copy(idx_hbm.at[pl.ds(base, b_per_w)], idx_v)

        def step(i, carry):
            off = pl.multiple_of(i * LANES, LANES)
            iv = idx_v[pl.ds(off, LANES)]
            v = plsc.load_gather(table_v, [iv])
            out_v[pl.ds(off, LANES)] = 1.0 / (v + EPS_DIV)
            return carry

        lax.fori_loop(0, b_per_w // LANES, step, 0)
        pltpu.sync_copy(out_v, out_hbm.at[0, pl.ds(base, b_per_w)])

    fn = pl.kernel(
        body,
        out_type=jax.ShapeDtypeStruct((1, g_pad), jnp.float32),
        mesh=mesh,
        compiler_params=pltpu.CompilerParams(needs_layout_passes=False),
        scratch_types=[
            pltpu.VMEM((scale_p.shape[0],), jnp.float32),
            pltpu.VMEM((b_per_w,), jnp.int32),
            pltpu.VMEM((b_per_w,), jnp.float32),
        ],
    )
    return fn(scale_p, idx_p)


def _tc_scale_mul(x_ng, recip_row, block_rows):
    n, g = x_ng.shape

    g_pad = recip_row.shape[1]

    def body(x_ref, r_ref, o_ref):
        o_ref[...] = x_ref[...] * r_ref[...][:, :g]

    return pl.pallas_call(
        body,
        grid=(pl.cdiv(n, block_rows),),
        in_specs=[
            pl.BlockSpec((block_rows, g), lambda i: (i, 0)),
            pl.BlockSpec((1, g_pad), lambda i: (0, 0)),
        ],
        out_specs=pl.BlockSpec((block_rows, g), lambda i: (i, 0)),
        out_shape=jax.ShapeDtypeStruct((n, g), jnp.float32),
    )(x_ng, recip_row)


@jax.jit
def kernel(x_ng, scale_g, idx):
    n, g = x_ng.shape
    chunk = NUM_WORKERS * LANES
    g_pad = ((g + chunk - 1) // chunk) * chunk
    idx_p = jnp.pad(idx, (0, g_pad - g))
    recip_row = _sc_recip_gather(scale_g, idx_p, g_pad)
    return _tc_scale_mul(x_ng, recip_row, block_rows=96)
